# baseline (speedup 1.0000x reference)
import functools

import jax
import jax.numpy as jnp
from jax import lax
from jax.experimental import pallas as pl
from jax.experimental.pallas import tpu as pltpu
from jax.experimental.pallas import tpu_sc as plsc

N = 50000
E = 1600000
IN = 128
HID = 32
OUT = 16
G = 8
K = 3
NB = G + K

NC = 2
NS = 16
NW = NC * NS

IW = 192
NBUF = 4
PER_W = 50688
NCHUNK = PER_W // IW
EPAD = PER_W * NW

NACC = 50176
RPS = NACC // NS
ZRD = 224
DW = 16


def _bases(x):
  u = (x + 1.75) * 4.0
  cf = jnp.floor(u)
  w = u - cf
  w2 = w * w
  w3 = w2 * w
  sixth = jnp.float32(1.0 / 6.0)
  b3 = w3 * sixth
  omw = 1.0 - w
  b0 = omw * omw * omw * sixth
  b2 = (1.0 + 3.0 * w + 3.0 * w2 - 3.0 * w3) * sixth
  b1 = (4.0 - 6.0 * w2 + 3.0 * w3) * sixth
  zero = jnp.zeros_like(x)
  out = []
  for j in range(NB):
    v = jnp.where(cf == float(j), b3, zero)
    v = v + jnp.where(cf == float(j + 1), b2, zero)
    v = v + jnp.where(cf == float(j + 2), b1, zero)
    v = v + jnp.where(cf == float(j + 3), b0, zero)
    out.append(v)
  return out


def _silu(x):
  return x * (1.0 / (1.0 + jnp.exp(-x)))


def _kan_pair(x, w1, b1t, w2, b2t):
  bs = _bases(x)
  x2 = jnp.dot(_silu(x), b1t, preferred_element_type=jnp.float32)
  for b in range(NB):
    x2 = x2 + jnp.dot(bs[b], w1[b], preferred_element_type=jnp.float32)
  bs2 = _bases(x2)
  acc = jnp.dot(_silu(x2), b2t, preferred_element_type=jnp.float32)
  for b in range(NB):
    acc = acc + jnp.dot(bs2[b], w2[b], preferred_element_type=jnp.float32)
  return acc


BN1 = 1000


def _kan1_body(h_ref, w1_ref, b1t_ref, w2_ref, b2t_ref, o_ref):
  o_ref[...] = _kan_pair(h_ref[...], w1_ref[...], b1t_ref[...], w2_ref[...],
                         b2t_ref[...])


def _kan1(h, w1, b1t, w2, b2t):
  grid = N // BN1
  return pl.pallas_call(
      _kan1_body,
      out_shape=jax.ShapeDtypeStruct((N, HID), jnp.float32),
      grid=(grid,),
      in_specs=[
          pl.BlockSpec((BN1, IN), lambda i: (i, 0)),
          pl.BlockSpec((NB, IN, 5), lambda i: (0, 0, 0)),
          pl.BlockSpec((IN, 5), lambda i: (0, 0)),
          pl.BlockSpec((NB, 5, HID), lambda i: (0, 0, 0)),
          pl.BlockSpec((5, HID), lambda i: (0, 0)),
      ],
      out_specs=pl.BlockSpec((BN1, HID), lambda i: (i, 0)),
  )(h, w1, b1t, w2, b2t)


def _deg_body(dstm_hbm, out_hbm, di_v, ones_v, zb_v, acc_sh):
  c = lax.axis_index("c")
  s = lax.axis_index("s")
  wid = c * NS + s

  @pl.loop(0, IW)
  def _fill(r):
    ones_v[r] = jnp.full((DW,), 1.0, jnp.float32)

  @pl.loop(0, ZRD)
  def _zfill(r):
    zb_v[r] = jnp.zeros((DW,), jnp.float32)

  @pl.loop(0, RPS, step=ZRD)
  def _zero(r0):
    pltpu.sync_copy(zb_v, acc_sh.at[pl.ds(s * RPS + r0, ZRD)])

  plsc.subcore_barrier()

  base = wid * PER_W

  @pl.loop(0, NCHUNK)
  def _chunk(ci):
    pltpu.sync_copy(dstm_hbm.at[pl.ds(base + ci * IW, IW)], di_v)
    pltpu.sync_copy(ones_v, acc_sh.at[di_v], add=True)

  plsc.subcore_barrier()

  @pl.loop(0, RPS, step=ZRD)
  def _wb(r0):
    pltpu.sync_copy(acc_sh.at[pl.ds(s * RPS + r0, ZRD)], zb_v)
    pltpu.sync_copy(zb_v, out_hbm.at[c, pl.ds(s * RPS + r0, ZRD)])


def _agg_body(h_hbm, srcm_hbm, dstm_hbm, out_hbm, si_v, di_v, rows_v,
              acc_sh, sem0, sem1, sem2, sem3):
  c = lax.axis_index("c")
  s = lax.axis_index("s")
  wid = c * NS + s
  sems = (sem0, sem1, sem2, sem3)

  @pl.loop(0, IW)
  def _zfill(r):
    @pl.loop(0, HID, step=16)
    def _zfill2(cc):
      rows_v[0, r, pl.ds(cc, 16)] = jnp.zeros((16,), jnp.float32)

  rem = RPS - (RPS // IW) * IW

  @pl.loop(0, RPS - rem, step=IW)
  def _zero(r0):
    pltpu.sync_copy(rows_v.at[0], acc_sh.at[pl.ds(s * RPS + r0, IW)])

  pltpu.sync_copy(rows_v.at[0, pl.ds(0, rem)],
                  acc_sh.at[pl.ds(s * RPS + RPS - rem, rem)])

  plsc.subcore_barrier()

  base = wid * PER_W

  def load_idx(ci, b):
    pltpu.sync_copy(srcm_hbm.at[pl.ds(base + ci * IW, IW)], si_v.at[b])
    pltpu.sync_copy(dstm_hbm.at[pl.ds(base + ci * IW, IW)], di_v.at[b])

  def fire(b):
    pltpu.make_async_copy(h_hbm.at[si_v.at[b]], rows_v.at[b], sems[b]).start()

  def drain(b):
    pltpu.make_async_copy(h_hbm.at[si_v.at[b]], rows_v.at[b], sems[b]).wait()

  def scatter(b):
    pltpu.sync_copy(rows_v.at[b], acc_sh.at[di_v.at[b]], add=True)

  for k in range(NBUF - 1):
    load_idx(k, k)
    fire(k)

  @pl.loop(0, NCHUNK, step=NBUF)
  def _chunk(ci0):
    for b in range(NBUF):
      ci = ci0 + b
      pf = (b + NBUF - 1) % NBUF

      @pl.when(ci + NBUF - 1 < NCHUNK)
      def _prefetch():
        load_idx(ci + NBUF - 1, pf)
        fire(pf)

      drain(b)
      scatter(b)

  plsc.subcore_barrier()

  @pl.loop(0, RPS, step=IW)
  def _wb(r0):
    @pl.when(r0 + IW <= RPS)
    def _full():
      pltpu.sync_copy(acc_sh.at[pl.ds(s * RPS + r0, IW)], rows_v.at[0])
      pltpu.sync_copy(rows_v.at[0], out_hbm.at[c, pl.ds(s * RPS + r0, IW)])

    @pl.when(r0 + IW > RPS)
    def _part():
      pltpu.sync_copy(acc_sh.at[pl.ds(s * RPS + r0, rem)],
                      rows_v.at[0, pl.ds(0, rem)])
      pltpu.sync_copy(rows_v.at[0, pl.ds(0, rem)],
                      out_hbm.at[c, pl.ds(s * RPS + r0, rem)])


@functools.lru_cache(maxsize=None)
def _sc_kernels():
  mesh = plsc.VectorSubcoreMesh(
      core_axis_name="c", subcore_axis_name="s", num_cores=NC, num_subcores=NS)
  cp = pltpu.CompilerParams(use_tc_tiling_on_sc=False)
  deg = pl.kernel(
      _deg_body,
      out_type=jax.ShapeDtypeStruct((NC, NACC, DW), jnp.float32),
      mesh=mesh,
      scratch_types=[
          pltpu.VMEM((IW,), jnp.int32),
          pltpu.VMEM((IW, DW), jnp.float32),
          pltpu.VMEM((ZRD, DW), jnp.float32),
          pltpu.VMEM_SHARED((NACC, DW), jnp.float32),
      ],
      compiler_params=cp,
  )
  agg = pl.kernel(
      _agg_body,
      out_type=jax.ShapeDtypeStruct((NC, NACC, HID), jnp.float32),
      mesh=mesh,
      scratch_types=[
          pltpu.VMEM((NBUF, IW), jnp.int32),
          pltpu.VMEM((NBUF, IW), jnp.int32),
          pltpu.VMEM((NBUF, IW, HID), jnp.float32),
          pltpu.VMEM_SHARED((NACC, HID), jnp.float32),
          pltpu.SemaphoreType.DMA,
          pltpu.SemaphoreType.DMA,
          pltpu.SemaphoreType.DMA,
          pltpu.SemaphoreType.DMA,
      ],
      compiler_params=cp,
  )
  return deg, agg


BN2 = 2000


def _sage_body(h_ref, ag_ref, dg_ref, wst_ref, wnt_ref, o_ref):
  hh = h_ref[...]
  a = ag_ref[0] + ag_ref[1]
  d = dg_ref[0, :, 0:1] + dg_ref[1, :, 0:1]
  agg = a * (1.0 / jnp.maximum(d, 1.0))
  m = (jnp.dot(hh, wst_ref[...], preferred_element_type=jnp.float32)
       + jnp.dot(agg, wnt_ref[...], preferred_element_type=jnp.float32)
       + hh)
  o_ref[...] = jnp.where(m >= 0, m, 0.01 * m)


def _sage(h1, aggp, degp, wst, wnt):
  grid = N // BN2
  return pl.pallas_call(
      _sage_body,
      out_shape=jax.ShapeDtypeStruct((N, HID), jnp.float32),
      grid=(grid,),
      in_specs=[
          pl.BlockSpec((BN2, HID), lambda i: (i, 0)),
          pl.BlockSpec((NC, BN2, HID), lambda i: (0, i, 0)),
          pl.BlockSpec((NC, BN2, DW), lambda i: (0, i, 0)),
          pl.BlockSpec((HID, HID), lambda i: (0, 0)),
          pl.BlockSpec((HID, HID), lambda i: (0, 0)),
      ],
      out_specs=pl.BlockSpec((BN2, HID), lambda i: (i, 0)),
  )(h1, aggp, degp, wst, wnt)


def _final_body(h_ref, ag_ref, dg_ref, wst_ref, wnt_ref, w3_ref, b3t_ref,
                w4_ref, b4t_ref, o_ref, ysum):
  i = pl.program_id(0)
  hh = h_ref[...]
  a = ag_ref[0] + ag_ref[1]
  d = dg_ref[0, :, 0:1] + dg_ref[1, :, 0:1]
  agg = a * (1.0 / jnp.maximum(d, 1.0))
  m = (jnp.dot(hh, wst_ref[...], preferred_element_type=jnp.float32)
       + jnp.dot(agg, wnt_ref[...], preferred_element_type=jnp.float32)
       + hh)
  h3 = jnp.where(m >= 0, m, 0.01 * m)
  part = jnp.sum(h3, axis=0, keepdims=True)

  @pl.when(i == 0)
  def _init():
    ysum[...] = part

  @pl.when(i > 0)
  def _acc():
    ysum[...] = ysum[...] + part

  @pl.when(i == N // BN2 - 1)
  def _readout():
    y = ysum[...]
    r = _kan_pair(y, w3_ref[...], b3t_ref[...], w4_ref[...], b4t_ref[...])
    o_ref[...] = 1.0 / (1.0 + jnp.exp(-r))


def _final(h2, aggp, degp, wst, wnt, w3, b3t, w4, b4t):
  grid = N // BN2
  return pl.pallas_call(
      _final_body,
      out_shape=jax.ShapeDtypeStruct((1, OUT), jnp.float32),
      grid=(grid,),
      in_specs=[
          pl.BlockSpec((BN2, HID), lambda i: (i, 0)),
          pl.BlockSpec((NC, BN2, HID), lambda i: (0, i, 0)),
          pl.BlockSpec((NC, BN2, DW), lambda i: (0, i, 0)),
          pl.BlockSpec((HID, HID), lambda i: (0, 0)),
          pl.BlockSpec((HID, HID), lambda i: (0, 0)),
          pl.BlockSpec((NB, HID, 5), lambda i: (0, 0, 0)),
          pl.BlockSpec((HID, 5), lambda i: (0, 0)),
          pl.BlockSpec((NB, 5, OUT), lambda i: (0, 0, 0)),
          pl.BlockSpec((5, OUT), lambda i: (0, 0)),
      ],
      out_specs=pl.BlockSpec((1, OUT), lambda i: (0, 0)),
      scratch_shapes=[pltpu.VMEM((1, HID), jnp.float32)],
  )(h2, aggp, degp, wst, wnt, w3, b3t, w4, b4t)


def kernel(h, edge_index, k1c1, k1b1, k1c2, k1b2, s1ws, s1wn, s2ws, s2wn,
           k2c1, k2b1, k2c2, k2b2):
  src = edge_index[0].astype(jnp.int32)
  dst = edge_index[1].astype(jnp.int32)
  pad = EPAD - E
  srcm = jnp.concatenate([src, jnp.zeros((pad,), jnp.int32)])
  dstm = jnp.concatenate([dst, jnp.full((pad,), N, jnp.int32)])

  w1 = jnp.transpose(k1c1, (2, 1, 0))
  b1t = k1b1.T
  w2 = jnp.transpose(k1c2, (2, 1, 0))
  b2t = k1b2.T
  w3 = jnp.transpose(k2c1, (2, 1, 0))
  b3t = k2b1.T
  w4 = jnp.transpose(k2c2, (2, 1, 0))
  b4t = k2b2.T

  deg_sc, agg_sc = _sc_kernels()
  srcm, dstm, h = lax.optimization_barrier((srcm, dstm, h))
  degp = deg_sc(dstm)
  h1 = _kan1(h, w1, b1t, w2, b2t)
  h1, degp = lax.optimization_barrier((h1, degp))
  aggp1 = agg_sc(h1, srcm, dstm)
  h2 = _sage(h1, aggp1, degp, s1ws.T, s1wn.T)
  aggp2 = agg_sc(h2, srcm, dstm)
  return _final(h2, aggp2, degp, s2ws.T, s2wn.T, w3, b3t, w4, b4t)


# speedup vs baseline: 1.2526x; 1.2526x over previous
import functools

import jax
import jax.numpy as jnp
from jax import lax
from jax.experimental import pallas as pl
from jax.experimental.pallas import tpu as pltpu
from jax.experimental.pallas import tpu_sc as plsc

N = 50000
E = 1600000
IN = 128
HID = 32
OUT = 16
G = 8
K = 3
NB = G + K

NC = 2
NS = 16
NW = NC * NS

KST = 2
CH = KST * 128
PER_W = 50176
NCHUNK = PER_W // CH
EPAD = PER_W * NW
IDXROWS = EPAD // 128
ROWS_PER_W = PER_W // 128

NACC = 50176
RPS = NACC // NS
ZR = 224
DW = 16


def _bases(x):
  u = (x + 1.75) * 4.0
  cf = jnp.floor(u)
  w = u - cf
  w2 = w * w
  w3 = w2 * w
  sixth = jnp.float32(1.0 / 6.0)
  b3 = w3 * sixth
  omw = 1.0 - w
  b0 = omw * omw * omw * sixth
  b2 = (1.0 + 3.0 * w + 3.0 * w2 - 3.0 * w3) * sixth
  b1 = (4.0 - 6.0 * w2 + 3.0 * w3) * sixth
  zero = jnp.zeros_like(x)
  out = []
  for j in range(NB):
    v = jnp.where(cf == float(j), b3, zero)
    v = v + jnp.where(cf == float(j + 1), b2, zero)
    v = v + jnp.where(cf == float(j + 2), b1, zero)
    v = v + jnp.where(cf == float(j + 3), b0, zero)
    out.append(v)
  return out


def _silu(x):
  return x * (1.0 / (1.0 + jnp.exp(-x)))


def _kan_pair(x, w1, b1t, w2, b2t):
  bs = _bases(x)
  x2 = jnp.dot(_silu(x), b1t, preferred_element_type=jnp.float32)
  for b in range(NB):
    x2 = x2 + jnp.dot(bs[b], w1[b], preferred_element_type=jnp.float32)
  bs2 = _bases(x2)
  acc = jnp.dot(_silu(x2), b2t, preferred_element_type=jnp.float32)
  for b in range(NB):
    acc = acc + jnp.dot(bs2[b], w2[b], preferred_element_type=jnp.float32)
  return acc


BN1 = 1000


def _kan1_body(h_ref, w1_ref, b1t_ref, w2_ref, b2t_ref, o_ref):
  o_ref[...] = _kan_pair(h_ref[...], w1_ref[...], b1t_ref[...], w2_ref[...],
                         b2t_ref[...])


def _kan1(h, w1, b1t, w2, b2t):
  grid = N // BN1
  return pl.pallas_call(
      _kan1_body,
      out_shape=jax.ShapeDtypeStruct((N, HID), jnp.float32),
      grid=(grid,),
      in_specs=[
          pl.BlockSpec((BN1, IN), lambda i: (i, 0)),
          pl.BlockSpec((NB, IN, 5), lambda i: (0, 0, 0)),
          pl.BlockSpec((IN, 5), lambda i: (0, 0)),
          pl.BlockSpec((NB, 5, HID), lambda i: (0, 0, 0)),
          pl.BlockSpec((5, HID), lambda i: (0, 0)),
      ],
      out_specs=pl.BlockSpec((BN1, HID), lambda i: (i, 0)),
  )(h, w1, b1t, w2, b2t)


def _deg_body(dstm_hbm, out_hbm, di_v, ones_v, zb_v, acc_sh):
  c = lax.axis_index("c")
  s = lax.axis_index("s")
  wid = c * NS + s

  @pl.loop(0, 128)
  def _fill(r):
    ones_v[r] = jnp.full((DW,), 1.0, jnp.float32)

  @pl.loop(0, ZR)
  def _zfill(r):
    zb_v[r] = jnp.zeros((DW,), jnp.float32)

  @pl.loop(0, RPS, step=ZR)
  def _zero(r0):
    pltpu.sync_copy(zb_v, acc_sh.at[pl.ds(s * RPS + r0, ZR)])

  plsc.subcore_barrier()

  @pl.loop(0, NCHUNK)
  def _chunk(ci):
    br = wid * ROWS_PER_W + ci * KST
    pltpu.sync_copy(dstm_hbm.at[pl.ds(br, KST)], di_v)
    for j in range(KST):
      pltpu.sync_copy(ones_v, acc_sh.at[di_v.at[j]], add=True)

  plsc.subcore_barrier()

  @pl.loop(0, RPS, step=ZR)
  def _wb(r0):
    pltpu.sync_copy(acc_sh.at[pl.ds(s * RPS + r0, ZR)], zb_v)
    pltpu.sync_copy(zb_v, out_hbm.at[c, pl.ds(s * RPS + r0, ZR)])


def _agg_body(h_hbm, srcm_hbm, dstm_hbm, out_hbm, si_v, di_v, rows_v, zb_v,
              acc_sh, sem0, sem1):
  c = lax.axis_index("c")
  s = lax.axis_index("s")
  wid = c * NS + s
  sems = (sem0, sem1)

  @pl.loop(0, ZR)
  def _zfill(r):
    @pl.loop(0, HID, step=16)
    def _zfill2(cc):
      zb_v[r, pl.ds(cc, 16)] = jnp.zeros((16,), jnp.float32)

  @pl.loop(0, RPS, step=ZR)
  def _zero(r0):
    pltpu.sync_copy(zb_v, acc_sh.at[pl.ds(s * RPS + r0, ZR)])

  plsc.subcore_barrier()

  base = wid * ROWS_PER_W

  def load_idx(ci, b):
    pltpu.sync_copy(srcm_hbm.at[pl.ds(base + ci * KST, KST)], si_v.at[b])
    pltpu.sync_copy(dstm_hbm.at[pl.ds(base + ci * KST, KST)], di_v.at[b])

  def fire(b):
    for j in range(KST):
      pltpu.make_async_copy(
          h_hbm.at[si_v.at[b, j]], rows_v.at[b, j], sems[b]).start()

  def drain(b):
    for j in range(KST):
      pltpu.make_async_copy(
          h_hbm.at[si_v.at[b, j]], rows_v.at[b, j], sems[b]).wait()

  def scatter(b):
    for j in range(KST):
      pltpu.sync_copy(rows_v.at[b, j], acc_sh.at[di_v.at[b, j]], add=True)

  load_idx(0, 0)
  fire(0)

  @pl.loop(0, NCHUNK, step=2)
  def _chunk(ci0):
    for b in (0, 1):
      ci = ci0 + b
      nb = 1 - b

      @pl.when(ci + 1 < NCHUNK)
      def _prefetch():
        load_idx(ci + 1, nb)
        fire(nb)

      drain(b)
      scatter(b)

  plsc.subcore_barrier()

  @pl.loop(0, RPS, step=ZR)
  def _wb(r0):
    pltpu.sync_copy(acc_sh.at[pl.ds(s * RPS + r0, ZR)], zb_v)
    pltpu.sync_copy(zb_v, out_hbm.at[c, pl.ds(s * RPS + r0, ZR)])


@functools.lru_cache(maxsize=None)
def _sc_kernels():
  mesh = plsc.VectorSubcoreMesh(
      core_axis_name="c", subcore_axis_name="s", num_cores=NC, num_subcores=NS)
  cp = pltpu.CompilerParams(use_tc_tiling_on_sc=False)
  deg = pl.kernel(
      _deg_body,
      out_type=jax.ShapeDtypeStruct((NC, NACC, DW), jnp.float32),
      mesh=mesh,
      scratch_types=[
          pltpu.VMEM((KST, 128), jnp.int32),
          pltpu.VMEM((128, DW), jnp.float32),
          pltpu.VMEM((ZR, DW), jnp.float32),
          pltpu.VMEM_SHARED((NACC, DW), jnp.float32),
      ],
      compiler_params=cp,
  )
  agg = pl.kernel(
      _agg_body,
      out_type=jax.ShapeDtypeStruct((NC, NACC, HID), jnp.float32),
      mesh=mesh,
      scratch_types=[
          pltpu.VMEM((2, KST, 128), jnp.int32),
          pltpu.VMEM((2, KST, 128), jnp.int32),
          pltpu.VMEM((2, KST, 128, HID), jnp.float32),
          pltpu.VMEM((ZR, HID), jnp.float32),
          pltpu.VMEM_SHARED((NACC, HID), jnp.float32),
          pltpu.SemaphoreType.DMA,
          pltpu.SemaphoreType.DMA,
      ],
      compiler_params=cp,
  )
  return deg, agg


BN2 = 2000


def _sage_body(h_ref, ag_ref, dg_ref, wst_ref, wnt_ref, o_ref):
  hh = h_ref[...]
  a = ag_ref[0] + ag_ref[1]
  d = dg_ref[0, :, 0:1] + dg_ref[1, :, 0:1]
  agg = a * (1.0 / jnp.maximum(d, 1.0))
  m = (jnp.dot(hh, wst_ref[...], preferred_element_type=jnp.float32)
       + jnp.dot(agg, wnt_ref[...], preferred_element_type=jnp.float32)
       + hh)
  o_ref[...] = jnp.where(m >= 0, m, 0.01 * m)


def _sage(h1, aggp, degp, wst, wnt):
  grid = N // BN2
  return pl.pallas_call(
      _sage_body,
      out_shape=jax.ShapeDtypeStruct((N, HID), jnp.float32),
      grid=(grid,),
      in_specs=[
          pl.BlockSpec((BN2, HID), lambda i: (i, 0)),
          pl.BlockSpec((NC, BN2, HID), lambda i: (0, i, 0)),
          pl.BlockSpec((NC, BN2, DW), lambda i: (0, i, 0)),
          pl.BlockSpec((HID, HID), lambda i: (0, 0)),
          pl.BlockSpec((HID, HID), lambda i: (0, 0)),
      ],
      out_specs=pl.BlockSpec((BN2, HID), lambda i: (i, 0)),
  )(h1, aggp, degp, wst, wnt)


def _final_body(h_ref, ag_ref, dg_ref, wst_ref, wnt_ref, w3_ref, b3t_ref,
                w4_ref, b4t_ref, o_ref, ysum):
  i = pl.program_id(0)
  hh = h_ref[...]
  a = ag_ref[0] + ag_ref[1]
  d = dg_ref[0, :, 0:1] + dg_ref[1, :, 0:1]
  agg = a * (1.0 / jnp.maximum(d, 1.0))
  m = (jnp.dot(hh, wst_ref[...], preferred_element_type=jnp.float32)
       + jnp.dot(agg, wnt_ref[...], preferred_element_type=jnp.float32)
       + hh)
  h3 = jnp.where(m >= 0, m, 0.01 * m)
  part = jnp.sum(h3, axis=0, keepdims=True)

  @pl.when(i == 0)
  def _init():
    ysum[...] = part

  @pl.when(i > 0)
  def _acc():
    ysum[...] = ysum[...] + part

  @pl.when(i == N // BN2 - 1)
  def _readout():
    y = ysum[...]
    r = _kan_pair(y, w3_ref[...], b3t_ref[...], w4_ref[...], b4t_ref[...])
    o_ref[...] = 1.0 / (1.0 + jnp.exp(-r))


def _final(h2, aggp, degp, wst, wnt, w3, b3t, w4, b4t):
  grid = N // BN2
  return pl.pallas_call(
      _final_body,
      out_shape=jax.ShapeDtypeStruct((1, OUT), jnp.float32),
      grid=(grid,),
      in_specs=[
          pl.BlockSpec((BN2, HID), lambda i: (i, 0)),
          pl.BlockSpec((NC, BN2, HID), lambda i: (0, i, 0)),
          pl.BlockSpec((NC, BN2, DW), lambda i: (0, i, 0)),
          pl.BlockSpec((HID, HID), lambda i: (0, 0)),
          pl.BlockSpec((HID, HID), lambda i: (0, 0)),
          pl.BlockSpec((NB, HID, 5), lambda i: (0, 0, 0)),
          pl.BlockSpec((HID, 5), lambda i: (0, 0)),
          pl.BlockSpec((NB, 5, OUT), lambda i: (0, 0, 0)),
          pl.BlockSpec((5, OUT), lambda i: (0, 0)),
      ],
      out_specs=pl.BlockSpec((1, OUT), lambda i: (0, 0)),
      scratch_shapes=[pltpu.VMEM((1, HID), jnp.float32)],
  )(h2, aggp, degp, wst, wnt, w3, b3t, w4, b4t)


def kernel(h, edge_index, k1c1, k1b1, k1c2, k1b2, s1ws, s1wn, s2ws, s2wn,
           k2c1, k2b1, k2c2, k2b2):
  src = edge_index[0].astype(jnp.int32)
  dst = edge_index[1].astype(jnp.int32)
  pad = EPAD - E
  srcm = jnp.concatenate([src, jnp.zeros((pad,), jnp.int32)]).reshape(
      IDXROWS, 128)
  dstm = jnp.concatenate([dst, jnp.full((pad,), N, jnp.int32)]).reshape(
      IDXROWS, 128)

  w1 = jnp.transpose(k1c1, (2, 1, 0))
  b1t = k1b1.T
  w2 = jnp.transpose(k1c2, (2, 1, 0))
  b2t = k1b2.T
  w3 = jnp.transpose(k2c1, (2, 1, 0))
  b3t = k2b1.T
  w4 = jnp.transpose(k2c2, (2, 1, 0))
  b4t = k2b2.T

  deg_sc, agg_sc = _sc_kernels()
  srcm, dstm, h = lax.optimization_barrier((srcm, dstm, h))
  degp = deg_sc(dstm)
  h1 = _kan1(h, w1, b1t, w2, b2t)
  h1, degp = lax.optimization_barrier((h1, degp))
  aggp1 = agg_sc(h1, srcm, dstm)
  h2 = _sage(h1, aggp1, degp, s1ws.T, s1wn.T)
  aggp2 = agg_sc(h2, srcm, dstm)
  return _final(h2, aggp2, degp, s2ws.T, s2wn.T, w3, b3t, w4, b4t)
